# baseline (speedup 1.0000x reference)
import jax
import jax.numpy as jnp
from jax.experimental import pallas as pl
from jax.experimental.pallas import tpu as pltpu

def _wr_kernel(s_ref, out_ref):
    out_ref[...] = s_ref[0, 0] + jnp.zeros_like(out_ref)

def _rd_kernel(x_ref, acc_ref, out_ref):
    i = pl.program_id(0)
    @pl.when(i == 0)
    def _():
        acc_ref[...] = jnp.zeros_like(acc_ref)
    acc_ref[...] += x_ref[...] * 1e-9
    @pl.when(i == pl.num_programs(0) - 1)
    def _():
        out_ref[...] = acc_ref[...]

def kernel(x, protos, log_alpha, log_sigma):
    B, T, D = x.shape
    n = B * T
    x2 = x.reshape(n, D)
    block = 2048
    s = jnp.reshape(log_alpha, (1, 1))
    wr = pl.pallas_call(
        _wr_kernel,
        grid=(n // block,),
        in_specs=[pl.BlockSpec(memory_space=pltpu.SMEM)],
        out_specs=pl.BlockSpec((block, D), lambda i: (i, 0)),
        out_shape=jax.ShapeDtypeStruct((n, D), jnp.float32),
    )(s)
    rd = pl.pallas_call(
        _rd_kernel,
        grid=(n // block,),
        in_specs=[pl.BlockSpec((block, D), lambda i: (i, 0))],
        out_specs=pl.BlockSpec((block, D), lambda i: (0, 0)),
        out_shape=jax.ShapeDtypeStruct((block, D), jnp.float32),
        scratch_shapes=[pltpu.VMEM((block, D), jnp.float32)],
    )(x2)
    out = wr + 1e-20 * jnp.sum(rd)
    return out.reshape(B, T, D), protos


# speedup vs baseline: 1.0141x; 1.0141x over previous
import math

import jax
import jax.numpy as jnp
from jax.experimental import pallas as pl
from jax.experimental.pallas import tpu as pltpu

K = 16
DECAY = 0.95
SQRT_2_OVER_PI = math.sqrt(2.0 / math.pi)

BLOCK = 512
NBLOCKS = 32768 // BLOCK
CACHED = 46
UNCACHED = NBLOCKS - CACHED


def _gelu(x):
    return 0.5 * x * (1.0 + jnp.tanh(SQRT_2_OVER_PI * (x + 0.044715 * x * x * x)))


def _row_normalize(v):
    n = jnp.sqrt(jnp.sum(v * v, axis=-1, keepdims=True))
    return v / jnp.maximum(n, 1e-12)


def _dot(a, b, dims, out_dtype=jnp.float32):
    return jax.lax.dot_general(a, b, (dims, ((), ())),
                               preferred_element_type=out_dtype)


def _fused_kernel(x_ref, protos_ref, la_ref, ls_ref,
                  out_ref, protos_out_ref,
                  cache_ref, pnb_ref, sums_ref, counts_ref, nov_ref, gate_ref,
                  *, n_tokens):
    i = pl.program_id(0)

    @pl.when(i == 0)
    def _init():
        sums_ref[...] = jnp.zeros_like(sums_ref)
        counts_ref[...] = jnp.zeros_like(counts_ref)
        nov_ref[0, 0] = 0.0
        pnb_ref[...] = _row_normalize(protos_ref[...]).astype(jnp.bfloat16)

    @pl.when(i < NBLOCKS)
    def _phase1():
        x = x_ref[...]
        xb = x.astype(jnp.bfloat16)

        @pl.when(i >= UNCACHED)
        def _store_cache():
            cache_ref[i - UNCACHED] = xb

        sim_t = _dot(pnb_ref[...], xb, (((1,), (1,))))
        norm2 = _dot(xb * xb, jnp.ones((xb.shape[1], 1), jnp.bfloat16),
                     (((1,), (0,))))
        inv_norm = jax.lax.rsqrt(jnp.maximum(norm2, 1e-24))

        m_t = jnp.max(sim_t, axis=0, keepdims=True)
        one_hot_t = (sim_t == m_t).astype(jnp.bfloat16)

        sums_ref[...] += _dot(one_hot_t, xb, (((1,), (0,))))
        counts_ref[...] += _dot(one_hot_t,
                                jnp.ones((BLOCK, 128), jnp.bfloat16),
                                (((1,), (0,))))
        nov_ref[0, 0] += _dot(m_t, inv_norm, (((1,), (0,))))[0, 0]

        @pl.when(i == NBLOCKS - 1)
        def _finalize():
            p = protos_ref[...]
            cnt = counts_ref[:, 0:1]
            cnt_kd = _dot(cnt, jnp.ones((1, p.shape[1]), jnp.float32),
                          (((1,), (0,))))
            centroid = sums_ref[...] / jnp.maximum(cnt_kd, 1.0)
            centroid = _row_normalize(centroid)
            upd = _row_normalize(DECAY * p + (1.0 - DECAY) * centroid)
            protos_out_ref[...] = jnp.where(cnt_kd > 0.0, upd, p)
            novelty = 1.0 - nov_ref[0, 0] / n_tokens
            alpha = jnp.exp(la_ref[0, 0])
            sigma = jnp.exp(ls_ref[0, 0])
            gate_ref[0, 0] = 1.0 + alpha * jnp.tanh(sigma * novelty)

    @pl.when(i >= NBLOCKS)
    def _phase2():
        j = i - NBLOCKS
        gate = gate_ref[0, 0]

        @pl.when(j < UNCACHED)
        def _from_hbm():
            out_ref[...] = _gelu(x_ref[...]) * gate

        @pl.when(j >= UNCACHED)
        def _from_cache():
            xc = cache_ref[j - UNCACHED].astype(jnp.float32)
            out_ref[...] = _gelu(xc) * gate


def _x_index(i):
    return (jnp.where(i < NBLOCKS, i,
                      jnp.minimum(i - NBLOCKS, UNCACHED - 1)), 0)


def kernel(x, protos, log_alpha, log_sigma):
    B, T, D = x.shape
    n_tokens = B * T
    x2 = x.reshape(n_tokens, D)

    la = jnp.reshape(log_alpha, (1, 1)).astype(jnp.float32)
    ls = jnp.reshape(log_sigma, (1, 1)).astype(jnp.float32)

    out, new_protos = pl.pallas_call(
        lambda *refs: _fused_kernel(*refs, n_tokens=n_tokens),
        grid=(2 * NBLOCKS,),
        in_specs=[
            pl.BlockSpec((BLOCK, D), _x_index),
            pl.BlockSpec((K, D), lambda i: (0, 0)),
            pl.BlockSpec(memory_space=pltpu.SMEM),
            pl.BlockSpec(memory_space=pltpu.SMEM),
        ],
        out_specs=[
            pl.BlockSpec((BLOCK, D), lambda i: (jnp.maximum(i - NBLOCKS, 0), 0)),
            pl.BlockSpec((K, D), lambda i: (0, 0)),
        ],
        out_shape=[
            jax.ShapeDtypeStruct((n_tokens, D), jnp.float32),
            jax.ShapeDtypeStruct((K, D), jnp.float32),
        ],
        scratch_shapes=[
            pltpu.VMEM((CACHED, BLOCK, D), jnp.bfloat16),
            pltpu.VMEM((K, D), jnp.bfloat16),
            pltpu.VMEM((K, D), jnp.float32),
            pltpu.VMEM((K, 128), jnp.float32),
            pltpu.SMEM((1, 1), jnp.float32),
            pltpu.SMEM((1, 1), jnp.float32),
        ],
    )(x2, protos, la, ls)

    return out.reshape(B, T, D), new_protos


# speedup vs baseline: 1.2359x; 1.2187x over previous
import math

import jax
import jax.numpy as jnp
from jax.experimental import pallas as pl
from jax.experimental.pallas import tpu as pltpu

K = 16
DECAY = 0.95
SQRT_2_OVER_PI = math.sqrt(2.0 / math.pi)

BLOCK1 = 4096
BLOCK2 = 2048


def _gelu(x):
    return 0.5 * x * (1.0 + jnp.tanh(SQRT_2_OVER_PI * (x + 0.044715 * x * x * x)))


def _row_normalize(v):
    n = jnp.sqrt(jnp.sum(v * v, axis=-1, keepdims=True))
    return v / jnp.maximum(n, 1e-12)


def _dot(a, b, dims, out_dtype=jnp.float32):
    return jax.lax.dot_general(a, b, (dims, ((), ())),
                               preferred_element_type=out_dtype)


def _pass1_kernel(x_ref, protos_ref, la_ref, ls_ref,
                  protos_out_ref, gate_ref,
                  sums_ref, counts_ref, nov_ref,
                  *, nblocks, n_tokens):
    i = pl.program_id(0)

    @pl.when(i == 0)
    def _init():
        sums_ref[...] = jnp.zeros_like(sums_ref)
        counts_ref[...] = jnp.zeros_like(counts_ref)
        nov_ref[0, 0] = 0.0

    x = x_ref[...]
    xb = x.astype(jnp.bfloat16)
    pnb = _row_normalize(protos_ref[...]).astype(jnp.bfloat16)

    sim_t = _dot(pnb, xb, (((1,), (1,))))
    norm2 = _dot(xb * xb, jnp.ones((xb.shape[1], 1), jnp.bfloat16),
                 (((1,), (0,))))
    inv_norm = jax.lax.rsqrt(jnp.maximum(norm2, 1e-24))

    m_t = jnp.max(sim_t, axis=0, keepdims=True)
    one_hot_t = (sim_t == m_t).astype(jnp.bfloat16)

    sums_ref[...] += _dot(one_hot_t, xb, (((1,), (0,))))
    counts_ref[...] += _dot(one_hot_t,
                            jnp.ones((BLOCK1, 128), jnp.bfloat16),
                            (((1,), (0,))))
    nov_ref[0, 0] += _dot(m_t, inv_norm, (((1,), (0,))))[0, 0]

    @pl.when(i == nblocks - 1)
    def _finalize():
        p = protos_ref[...]
        cnt = counts_ref[:, 0:1]
        cnt_kd = _dot(cnt, jnp.ones((1, p.shape[1]), jnp.float32),
                      (((1,), (0,))))
        centroid = sums_ref[...] / jnp.maximum(cnt_kd, 1.0)
        centroid = _row_normalize(centroid)
        upd = _row_normalize(DECAY * p + (1.0 - DECAY) * centroid)
        protos_out_ref[...] = jnp.where(cnt_kd > 0.0, upd, p)
        novelty = 1.0 - nov_ref[0, 0] / n_tokens
        alpha = jnp.exp(la_ref[0, 0])
        sigma = jnp.exp(ls_ref[0, 0])
        gate_ref[0, 0] = 1.0 + alpha * jnp.tanh(sigma * novelty)


def _pass2_kernel(gate_ref, x_ref, out_ref):
    out_ref[...] = _gelu(x_ref[...]) * gate_ref[0, 0]


def kernel(x, protos, log_alpha, log_sigma):
    B, T, D = x.shape
    n_tokens = B * T
    x2 = x.reshape(n_tokens, D)

    la = jnp.reshape(log_alpha, (1, 1)).astype(jnp.float32)
    ls = jnp.reshape(log_sigma, (1, 1)).astype(jnp.float32)

    nb1 = n_tokens // BLOCK1
    new_protos, gate = pl.pallas_call(
        lambda *refs: _pass1_kernel(*refs, nblocks=nb1, n_tokens=n_tokens),
        grid=(nb1,),
        in_specs=[
            pl.BlockSpec((BLOCK1, D), lambda i: (i, 0)),
            pl.BlockSpec((K, D), lambda i: (0, 0)),
            pl.BlockSpec(memory_space=pltpu.SMEM),
            pl.BlockSpec(memory_space=pltpu.SMEM),
        ],
        out_specs=[
            pl.BlockSpec((K, D), lambda i: (0, 0)),
            pl.BlockSpec(memory_space=pltpu.SMEM),
        ],
        out_shape=[
            jax.ShapeDtypeStruct((K, D), jnp.float32),
            jax.ShapeDtypeStruct((1, 1), jnp.float32),
        ],
        scratch_shapes=[
            pltpu.VMEM((K, D), jnp.float32),
            pltpu.VMEM((K, 128), jnp.float32),
            pltpu.SMEM((1, 1), jnp.float32),
        ],
    )(x2, protos, la, ls)

    out = pl.pallas_call(
        _pass2_kernel,
        grid=(n_tokens // BLOCK2,),
        in_specs=[
            pl.BlockSpec(memory_space=pltpu.SMEM),
            pl.BlockSpec((BLOCK2, D), lambda i: (i, 0)),
        ],
        out_specs=pl.BlockSpec((BLOCK2, D), lambda i: (i, 0)),
        out_shape=jax.ShapeDtypeStruct((n_tokens, D), jnp.float32),
    )(gate, x2)

    return out.reshape(B, T, D), new_protos
